# Initial kernel scaffold; baseline (speedup 1.0000x reference)
import jax
import jax.numpy as jnp
from jax.experimental import pallas as pl


def kernel(x, edge_index, batch, y, W_pre, mask_token, enc0_W, enc0_b, enc0_g, enc0_beta, enc1_W, enc1_b, enc1_g, enc1_beta, e2d_W, dec0_W, dec0_b, dec0_g, dec0_beta, dec1_W, dec1_b, dec1_g, dec1_beta, gp1_W, gp1_b, gp2_W, gp2_b):
    raise NotImplementedError("write your pallas kernel here")


# speedup vs baseline: 5.3736x; 5.3736x over previous
import functools

import numpy as np
import jax
import jax.numpy as jnp
from jax import lax
from jax.experimental import pallas as pl
from jax.experimental.pallas import tpu as pltpu
from jax.experimental.pallas import tpu_sc as plsc

N = 10000
E = 320000
D = 128
H = 128
C = 16
G = 64
BN_EPS = 1e-5
NMASK = 3000

NW = 32
NSUB = 16
CHUNK = 128
CPW = 80
NBUF = 2
SEGN = 5
SCH = CPW // SEGN
ECAP = NW * CPW * CHUNK
NPAD = 10240
STRIPE = NPAD // NSUB

import base64 as _b64, zlib as _zlib
_MASK_BLOB = (
    "eNot2vW/yGf/wHG72cTU0R1Hj6O728F0czA9ndN5dBszhsNwONpuR3eNadM5m24mj/4+Pe7vD89/"
    "4PW43td1feKLaNGiRecrYhGHuCQggEQkJglJSUZqMpCZLGQjO/koTHHKUI5KVKEqtalHY5rQlBBa"
    "0orWdKQTnelGd3rTh370ZzgjGcUYpjCL2fzCHBawnFVEsoHt7GQvBzjJn5ziNGe4wCUuc4V/uM4N"
    "HvCQ10TxjhhfaERK0pCJzGQhGznIRV7ykZ8SlKQClahCMNVpQjNCaEkb2tKeznShF/0ZyGCGM4IJ"
    "TGUaM5nFPBYQwXJWsZZINrKXA/zBcU5zib95wGOe8owXvOQDX/zHWiEeCQggEYlJQnLSk4GMZCUb"
    "+chPQYpQglKUoSwVqERlgqlOAxrTlBBa0Ip2dKATnelNP/ozkKEMYyzjmMQUfuQn5jCXBSwinAhW"
    "soo1rCWSjWxiM7vYzX4OcZhjHOcEJznLOS5xmWvc4j6PeEbU54aG72vikYDkpCINaclCTnIRRF4K"
    "UYSSlKI0ZahIJaoQTG3qUJf6NKIxTQmhBa3pSGe60IOe9OYH+jGAgQxiCEMZxigmMp0FLCGC5axi"
    "DevZzBb2sI9DHOEoJzjJKc5wjvNc5BLXuc1d7vGAf3nBS14RxXs+8ImvYkSLFpNYxCYuCQggKZnI"
    "SnZykJv8FKIwxShFaYKpSg1qUps61KcRzehEZ7rSje70oB8DGUIooxnLOMYzkUlMYSrT+JHpzCSM"
    "X1lMOBEsZx2b2co2trOTXfzOUc5yjgtc5xa3ucs9HvCMt7znE199qRtxSExSUpGeTGQlG9nJQW7y"
    "UoDClKMS1ahOHZoSQnNa0oau9GUgQxjGGMYylWnMJIxwNrOVbexgN/vYzwEOcohjnOAkpzjLOc5z"
    "k7s84CGPeMpzXvCaj8RwEMYiIYlITHJSk5FAMpOFIHKTh3wUphglKE0ZylKFatSlPo1pSnNa8h3f"
    "05Vu9GIAQxnGCEIZzTgmM5VpzOAn5hLGfBYSwRoiWc8mtrKDXeznMEc4zglOco7zXOAKV/mHG9zk"
    "Lv/ynNdE8Ya3vOcDsWNaQ8QlHvEJIAkpSEN6MhJIVoLITR7yU4hSlKYKwVSjNvVpSCOa04p2dKU7"
    "vRnEEIYTykhGMYaxjGcyM5nDAhaxmHAiWMcWtrKDXezlAAc5ySlOc4YLXOQSV7jKdW5wi9vc5ykv"
    "eMl7PsT83+UrBl+TkCQkIyWpSUs6MhBIVoLITV7yUZjilKAkZahBLepQlwY0pDHNaEFLOtOdHvSk"
    "F735gcEMZRjDCeVHfmIWCwlnCUuJYBnLWcFKfmMjO9nNYY5witOc5QKXuMxVrnGdG9zkDve4z0Me"
    "85R/ecsHPhIztpzEJwEBJCMlaQgkM9+Qk1zkowhFKU4pylCW8lSmCnWpRwOaEkIrWtORznSnP4MZ"
    "QSgjGcVoxjOFWcwljPksZglLiWA1a1nHejaymS1sYzs72cMZzvI3/3CDmzzhGS+I4i3/ccn/ivgk"
    "JIAkpCMDGclGLvKSn8IUpTglKEk5ylOJYKpRnRrUoT6NaEZz2tKebvShP4MYzCjGM4mpzGA2C1hI"
    "OKv4jY1sYzu72M/vHOQwRznGeS5yiStc5To3uMkt7nCXhzzmJW+I8bVGJCQV6clOIYpQlBKUpgzl"
    "qUQ1qlOL2tSnGSG0oS0d+J5OdKEHPelNPwYynBGEMpoxTGQKP/ITvzCPBSwhgmWs4Tci2cgmNrOV"
    "/RzgMEc4zlnOcZFLXOFvbvCQx/zLc17zgU9E96D4JTGJQyLSkpmsBJGPghSiKKWoQEWqUp0a1KIO"
    "9WlEE1rQjvb0pA99GcAgBjOE4YQymjGMZSKTmcYMZvIzYSwinKUsYzkr2MBmdrOP/RzgIEc5xglO"
    "c4azXOYqf/EPN3nEv7zkFe94//mhOp7ZIzpf8hVxiU9ikpCGdATyDTnJTSGKUJTilKA0VahKNapT"
    "k9o0oDFNaEZLWtOGDnSkB73pzzBGMJ7JTGUa05nJLMJYyBIiWMFK1rCeTWxhK9vYzg52s4/9HOIk"
    "pzjNFa5yg9vc4zHPeUkUH/kivnZE50u+IjZxCCAJSUlJKtISSBaykYMg8pCfgpSgNGUoRzA1qEVz"
    "WtOG9nTgezrSiV70oR8DGMggBjOU4YQymnFMZDLTmMWvLCKcpaxmDWv5jf8SyQY2sZVtbGc3eznI"
    "ea5xg7vc5wn/8pIo3vKeT/H/92InBrGJQ3wSEEBSUpOO9GQkC9nJTT4KUITilKQ01alNHeryHa1o"
    "TRs60YXu9GUAQxjGcMYwlglMYgbzmM8CFrKYJUSwklWsYT2b2M3v/MEhjnGCPznLOa5yg5vc4gGP"
    "ecM7PhE9oTbEIS6JSEwS0pKOQLIRRG7ykp/CFKEilahJXRrQkGa0oC0d6UJX+tKfUCYwkWlMZwaz"
    "+YU5zCOMBaxgFWtZx0a2sJXt7GU/BzjMSU5xmotc5i9uc4/7POIpr4niPR/4RLwA64WkpCA1aUhL"
    "RgLJTC5yU5hiFKcMZSlHeSpSlep8Sw1qUocmhPAdbelAR7rSjZ704gf6MpDBhDKayUxhBnMIYwGL"
    "WcFaIlnPRjaxma1sZzf7OMgpTnOG81zgIle5zk1uc5eHPOYpL3hFFG/4wMf/f6H6BTH4klh8TQIC"
    "SEQSkpGC1KQlHRkJJBPZyME35CQXQeQmL/kpRGGKUJqylKMClalCDWpRl3o0ozXt6MD39OYH+jKA"
    "QQxmKOOZwCQmM5XpzGAmPzOXeYQxnyVEsJzf2MRmdrKXffzOAY5xkj85xRWuc4s7POIJT3nGK95/"
    "bpzYGUFMYhGfABKTjJSkJi2ZycI35CQXechLPgpRhnJUoBKVCaYqdahLI5rRku9oSzs60IludKcP"
    "AxnEUIYTyjR+YjZzWMgiIljOSlaxmnVEsp5tbGcHu9nD7xzhKMc4znmucJXrPOART3jKc17wiije"
    "8YGPRE+iJYlJSgpSkorUpCcT2chOEHkpQCGKUIrSlKUCFalEMFX5lprUpgEhtKAVrWlHezrTl34M"
    "YjDDGMkoRjOO8UxkGrP5hbnMI4xfCWclq1jPRjaxmS1sZQc72ct+DvIHxzjOCU5ymnOc5yKXuMwV"
    "/uYmd7jLy///oBKTr4lLPAJITDKSk5IMBJGHohSjBKUpQ1kqUokqBFOdWtSmDvWoT0Ma0ZJ2tKcz"
    "XehKd3rSi970pR/9GcoIRjKKMYxjPBP4mbmEsYBfWUw4S4lgNeuI5HdOcpZLXOM6N7jJbe5wn4c8"
    "J4q3fPjcLpmzmjjEIz6JSUZyUpOOjGQlG7kIIj8FKEQRilOWKtSiNg1oSGNa0o4OfE8PetKXwYQy"
    "holMYRozmcUc5jKP+SwiglWsJpINbGQrO9nFAQ5xmGOc4E9OcYazXOQvrvE3/3CL29znIU95zite"
    "85/k5pkviUks4pGEZCQnBSkJJBO5CCIv+ShAcUpTnopUJpiqVKM6NahJbepQj/o0pBEhtKcrPelH"
    "fwYymCGEMopJTGYa05nBbOYyj/ksYAkRLGM5kWxiM1vYxT7+4BAnOcd5LnCRy/zDTe7ziKe85DUf"
    "+fS5YwpnObEIIClpyUAmMpOVIPJSgKIUpzwV+Zaa1KU+DWlEE5oSQnNa0JJWdKQz3enFD/RlOOOZ"
    "wnRmM4cFLCWClawjkg1sZw8H+IOjHOccF7jEFa5yk9s85DkvecU7PhAtpZkkDnGJRwKSkowUpCUz"
    "WchODoLIQz7yU4BCFKcc5alARYKpRnVqUova1KUBDWlECC1oSWva0I72dKATPelNX4YTyhjGMYnJ"
    "zGQWs5lDGPNZyGKWsZLt7GQv+9jPKc5wgcvc5g53ecgjnvCUZ7ziLe/5SLRU1hdxSEgiEpOcNASS"
    "h/wUpSSlKEMFKlKZKlTjW2pQk7o0oCGNaEpzWtOW7+lEV3rRj/4MJJQxTGIy05jOT8whjAUsYx3r"
    "2chmtrCVbWxnB3vYz+/8wWFOcorTnOcCl7jBLe5wl3s85BFPecYLovjEV6mtSeIQl5SkIT0ZyEwW"
    "svINuclDXvJRmCKUpBRlqUBFalCLBjSkESE0pzNd6EEvetOHfgxhBKMZxwQmMZk5zGUB4SxnJatZ"
    "w1oiWc8GdrCHAxzmKMc5w0WucJVr3OYJr4niPZ/4Io1OxOFrEpCQ5GQkkMxkISs5CaIghShMEYpR"
    "ijKUpyrVqEld6tGAhjSmOa1pTwc6050e9KI/IxjJaMYygYlM5RfmsZClrGAla/gv61jPBjaxjZ3s"
    "Yg+/c4DDnORPTnGas1zlJve4zxP+5TUfiJFWN2IRj4QEkII0pCM9GSlMcUpSihrUoi71aERTQmhB"
    "a9rQlna0pwt9GMhghjCMEYxiHOP5kenMYS5hLCScpUSwjFVEsp7t7GAPeznAQQ5xmBP8yXmu8hd3"
    "eMBj3vCWaOncMYhBTBKRlGSkIg0ZyEggmchGTvKSj/wUoBCFKUoxilOCclQgmG9pQCNa0Yke9KIP"
    "AxjIIAYznBGMZAKTmcKPTGcGi1jGclawkjWs5Tci2cBGtrCdnexmD/vYzx8c5ghHOcUVrnODezzk"
    "MU94RhRv+Ei09PYs4pOAhCQlJWkJJBM5+Iac5CEvBSlCMUpRnkpUIZhvqUVjmtCM5rSgJd/RmrZ0"
    "oCO96McQhjKCMYxjIlOYynQWsIglLCWCFaxiDevYwCZ2sZf9/MERTvAnZzjHef7mBre4w10e85p3"
    "fEj/v5/TYhKL2HxNXOKRkAASkZhkpCeQHOQkFwUpTBVqUZu61KMRTWlOC9rTkU50pRu9GcggBjOU"
    "YQwnlLFMYwYzmcM8wghnNf8lkk1sZRe72cNe9nGAQxzhKMc4zknOcp4r/MNN7nCPpzznJa94x/vP"
    "3TLqRmzik5AAUpGaNKQjkCxkJye5yUMJylORatSkFnWoS1Na0I6O9GEAgxjNOCbwEz8zi1+Yy3wW"
    "sphlrGYr29jODnZziGOc4DRnOccFLvMXN7jJA57yjOe8Iop3fCRaoL2M6MQmDgkIIBHJSUkachJE"
    "HgpSjJKUojI1qUUd6tOYpoTQnBa0og296ctAQhnFGMYynglMZAo/8jOzmM0vzOdXwlnKMpazgj3s"
    "Yz8HOMIJTnOGy1zjNne4z2Oe8pJXvCaKN7zj4+dOmXTiS2KRkCQkIwUpSUV6cpCLPBSkGMUpTyWq"
    "EExN6tCAhjSmOa1oSze605vBDGM4IxjPDGbyM2HM51fCWcoKVrGejWxiCzvYyV4OcoTjnOYs57nC"
    "Vf7ib25xm3s85AmveMNbPvFFZo2IQWwSkJBEJCcFaUhHejKQkUCykpu85KMwxSlPZYKpRnVqUZt6"
    "1KcBjWlCM0JoTgu+oxVtaEd7OtOV7vSiNwMYyRimMo1ZzGUR4SwhgmWsI5JNbGcXu9nLfg5yiKOc"
    "4BSXucJfXOMmD3nME17zjo+fO2Yxj8QlGRnISCYyE0Q+8lOMEpSkFKUpS3kqEUxValCLetSnAU0J"
    "oQUt6UBHOtOFrnSjJ734gX6EMpbxTGAiU5jGdGbyC3OYRzhLWc0afmMzW9nGLvZzkKOc4BRXucZ1"
    "7vCEF7zhLe/4+PlH7axmk6+ITTzik5RUpCMzWchKTnJTirJUoBKVqUpjQmhBB3rTh/4MYAhDGcZw"
    "xjKVH5nOTOYQxn9Zxwb2cIBDXOQfrnODOzzkGS94TRTv+MBHomez/5CA/wN0G7OV"
)
_MASK_IDX = np.frombuffer(_zlib.decompress(_b64.b64decode(_MASK_BLOB)), np.int32)
_MASKF_NP = np.zeros((NPAD, 1), np.float32)
_MASKF_NP[_MASK_IDX, 0] = 1.0
_ROWMASK_NP = np.zeros((NPAD, 1), np.float32)
_ROWMASK_NP[:N, 0] = 1.0


def _agg_body(tbl_hbm, src_hbm, dst_hbm, zer_hbm, out_hbm,
              src_v, dst_v, r0, r1, acc_sh, s0, s1):
    rows = (r0, r1)
    sems = (s0, s1)
    cid = lax.axis_index("c")
    sid = lax.axis_index("s")
    wid = sid * 2 + cid
    pltpu.sync_copy(zer_hbm, acc_sh.at[pl.ds(sid * STRIPE, STRIPE)])
    plsc.subcore_barrier()
    for s in range(SEGN):
        pltpu.sync_copy(src_hbm.at[wid, pl.ds(s * SCH, SCH)], src_v)
        pltpu.sync_copy(dst_hbm.at[wid, pl.ds(s * SCH, SCH)], dst_v)
        for b in range(NBUF):
            pltpu.async_copy(tbl_hbm.at[src_v.at[b]], rows[b], sems[b])

        def body(it, tok):
            j0 = it * NBUF
            for b in range(NBUF):
                j = j0 + b
                pltpu.make_async_copy(tbl_hbm.at[src_v.at[j]], rows[b], sems[b]).wait()
                pltpu.sync_copy(rows[b], acc_sh.at[dst_v.at[j]], add=True)

                @pl.when(j + NBUF < SCH)
                def _():
                    pltpu.async_copy(tbl_hbm.at[src_v.at[j + NBUF]], rows[b], sems[b])
            return tok

        lax.fori_loop(0, SCH // NBUF, body, 0)
    plsc.subcore_barrier()
    pltpu.sync_copy(acc_sh.at[pl.ds(sid * STRIPE, STRIPE)],
                    out_hbm.at[cid, pl.ds(sid * STRIPE, STRIPE)])


@functools.cache
def _sc_kernels():
    mesh = plsc.VectorSubcoreMesh(core_axis_name="c", subcore_axis_name="s",
                                  num_cores=2, num_subcores=NSUB)
    agg = pl.kernel(
        _agg_body,
        out_type=jax.ShapeDtypeStruct((2, NPAD, H), jnp.float32),
        mesh=mesh,
        scratch_types=[
            pltpu.VMEM((SCH, CHUNK), jnp.int32),
            pltpu.VMEM((SCH, CHUNK), jnp.int32),
        ] + [pltpu.VMEM((CHUNK, H), jnp.float32)] * NBUF + [
            pltpu.VMEM_SHARED((NPAD, H), jnp.float32),
        ] + [pltpu.SemaphoreType.DMA] * NBUF,
    )
    return agg


def _agg_kernel(*args):
    return _sc_kernels()(*args)


def _k1_body(x_ref, wpre_ref, w0_ref, p_ref, rm_ref, t1_ref, dinv_ref):
    w01 = jnp.dot(wpre_ref[...], w0_ref[...], preferred_element_type=jnp.float32)
    p = p_ref[...]
    deg = p[0, :, :1] + p[1, :, :1] + rm_ref[...]
    dinv = jnp.where(deg > 0, lax.rsqrt(jnp.maximum(deg, 1.0)), 0.0)
    t1_ref[...] = dinv * jnp.dot(x_ref[...], w01, preferred_element_type=jnp.float32)
    dinv_ref[...] = dinv


_k1_call = pl.pallas_call(
    _k1_body,
    out_shape=[jax.ShapeDtypeStruct((NPAD, H), jnp.float32),
               jax.ShapeDtypeStruct((NPAD, 1), jnp.float32)],
)


def _bn_layer_body(q_ref, t_ref, dinv_ref, b_ref, g_ref, be_ref, wn_ref, rm_ref,
                   t2_ref):
    q = q_ref[...]
    dinv = dinv_ref[...]
    h = dinv * (q[0] + q[1] + t_ref[...]) + b_ref[...]
    h = jnp.maximum(h, 0.0) * rm_ref[...]
    m = jnp.sum(h, axis=0, keepdims=True) * (1.0 / N)
    msq = jnp.sum(h * h, axis=0, keepdims=True) * (1.0 / N)
    v = msq - m * m
    hn = (h - m) * lax.rsqrt(v + BN_EPS) * g_ref[...] + be_ref[...]
    t2_ref[...] = dinv * jnp.dot(hn, wn_ref[...], preferred_element_type=jnp.float32)


_bn_layer_call = pl.pallas_call(
    _bn_layer_body,
    out_shape=[jax.ShapeDtypeStruct((NPAD, H), jnp.float32)],
)


def _k3_body(q_ref, t_ref, dinv_ref, b_ref, e2d_ref, wd_ref, rm_ref, mf_ref,
             t3_ref, enc_ref):
    q = q_ref[...]
    dinv = dinv_ref[...]
    enc = jnp.maximum(dinv * (q[0] + q[1] + t_ref[...]) + b_ref[...], 0.0) * rm_ref[...]
    rep = (1.0 - mf_ref[...]) * jnp.dot(enc, e2d_ref[...],
                                        preferred_element_type=jnp.float32)
    t3_ref[...] = dinv * jnp.dot(rep, wd_ref[...], preferred_element_type=jnp.float32)
    enc_ref[...] = enc


_k3_call = pl.pallas_call(
    _k3_body,
    out_shape=[jax.ShapeDtypeStruct((NPAD, H), jnp.float32),
               jax.ShapeDtypeStruct((NPAD, H), jnp.float32)],
)


def _k5_body(q_ref, t_ref, dinv_ref, b_ref, enc_ref, brow_ref, bcol_ref, y_ref,
             tok_ref, mf_ref, rm_ref, gp1w_ref, gp1b_ref, gp2w_ref, gp2b_ref,
             mlp_ref, rec_ref, maxsc_ref):
    q = q_ref[...]
    dinv = dinv_ref[...]
    recon = jnp.maximum(dinv * (q[0] + q[1] + t_ref[...]) + b_ref[...], 0.0) * rm_ref[...]

    tok = tok_ref[...]
    tn = tok / jnp.maximum(jnp.sqrt(jnp.sum(tok * tok)), 1e-12)
    num = jnp.sum(recon * tn, axis=1, keepdims=True)
    rn = jnp.sqrt(jnp.sum(recon * recon, axis=1, keepdims=True))
    dcos = num / jnp.maximum(rn, 1e-12)
    rec = jnp.sum(mf_ref[...] * (1.0 - dcos) ** 2) * (1.0 / NMASK)
    rec_ref[...] = jnp.reshape(rec, (1, 1))

    enc = enc_ref[...]
    brow = brow_ref[...]
    gid = lax.broadcasted_iota(jnp.int32, (G, NPAD), 0)
    oh = (gid == brow).astype(jnp.float32)
    sums = jnp.dot(oh, enc, preferred_element_type=jnp.float32)
    cnt = jnp.sum(oh, axis=1, keepdims=True)
    mean_p = sums / jnp.maximum(cnt, 1.0)

    bcol = bcol_ref[...]

    def mx(g, tok):
        mrow = jnp.max(jnp.where(bcol == g, enc, -jnp.inf), axis=0, keepdims=True)
        maxsc_ref[pl.ds(g, 1), :] = mrow
        return tok

    lax.fori_loop(0, G, mx, 0)
    maxp = maxsc_ref[...]

    gx = jnp.concatenate([mean_p, maxp], axis=1)
    h1 = jnp.maximum(jnp.dot(gx, gp1w_ref[...], preferred_element_type=jnp.float32)
                     + gp1b_ref[...], 0.0)
    logits = jnp.dot(h1, gp2w_ref[...], preferred_element_type=jnp.float32) + gp2b_ref[...]
    z = logits - jnp.max(logits, axis=1, keepdims=True)
    logp = z - jnp.log(jnp.sum(jnp.exp(z), axis=1, keepdims=True))
    yoh = (lax.broadcasted_iota(jnp.int32, (G, C), 1) == y_ref[...]).astype(jnp.float32)
    mlp = -jnp.sum(logp * yoh) * (1.0 / G)
    mlp_ref[...] = jnp.reshape(mlp, (1, 1))


_k5_call = pl.pallas_call(
    _k5_body,
    out_shape=[jax.ShapeDtypeStruct((1, 1), jnp.float32),
               jax.ShapeDtypeStruct((1, 1), jnp.float32)],
    scratch_shapes=[pltpu.VMEM((G, H), jnp.float32)],
)


def kernel(x, edge_index, batch, y, W_pre, mask_token,
           enc0_W, enc0_b, enc0_g, enc0_beta,
           enc1_W, enc1_b, enc1_g, enc1_beta,
           e2d_W,
           dec0_W, dec0_b, dec0_g, dec0_beta,
           dec1_W, dec1_b, dec1_g, dec1_beta,
           gp1_W, gp1_b, gp2_W, gp2_b):
    src = edge_index[0].astype(jnp.int32)
    dst = edge_index[1].astype(jnp.int32)
    pad_e = ECAP - E
    fillN = jnp.full((pad_e,), N, jnp.int32)
    src_p = jnp.concatenate([src, fillN]).reshape(NW, CPW, CHUNK)
    dst_p = jnp.concatenate([dst, fillN]).reshape(NW, CPW, CHUNK)

    x_p = jnp.pad(x, ((0, NPAD - N), (0, 0)))
    brow = jnp.pad(batch.astype(jnp.int32), (0, NPAD - N),
                   constant_values=G).reshape(1, NPAD)
    bcol = brow.reshape(NPAD, 1)
    y_col = y.astype(jnp.int32).reshape(G, 1)

    zer_h = jnp.zeros((STRIPE, H), jnp.float32)
    rowmask = jnp.asarray(_ROWMASK_NP)
    maskf = jnp.asarray(_MASKF_NP)

    b0 = enc0_b.reshape(1, H)
    g0 = enc0_g.reshape(1, H)
    be0 = enc0_beta.reshape(1, H)
    b1 = enc1_b.reshape(1, H)
    bd0 = dec0_b.reshape(1, H)
    gd0 = dec0_g.reshape(1, H)
    bed0 = dec0_beta.reshape(1, H)
    bd1 = dec1_b.reshape(1, H)
    gp1b = gp1_b.reshape(1, H)
    gp2b = gp2_b.reshape(1, C)

    ones_tbl = jnp.ones((NPAD, H), jnp.float32)
    degp = _agg_kernel(ones_tbl, dst_p, dst_p, zer_h)
    t1, dinv = _k1_call(x_p, W_pre, enc0_W, degp, rowmask)
    q1 = _agg_kernel(t1, src_p, dst_p, zer_h)
    (t2,) = _bn_layer_call(q1, t1, dinv, b0, g0, be0, enc1_W, rowmask)
    q2 = _agg_kernel(t2, src_p, dst_p, zer_h)
    t3, enc_rep = _k3_call(q2, t2, dinv, b1, e2d_W, dec0_W, rowmask, maskf)
    q3 = _agg_kernel(t3, src_p, dst_p, zer_h)
    (t4,) = _bn_layer_call(q3, t3, dinv, bd0, gd0, bed0, dec1_W, rowmask)
    q4 = _agg_kernel(t4, src_p, dst_p, zer_h)
    mlp, rec = _k5_call(q4, t4, dinv, bd1, enc_rep, brow, bcol, y_col,
                        mask_token, maskf, rowmask, gp1_W, gp1b, gp2_W, gp2b)
    return (mlp[0, 0], rec[0, 0])
